# baseline (speedup 1.0000x reference)
import functools

import jax
import jax.numpy as jnp
from jax import lax
from jax.experimental import pallas as pl
from jax.experimental.pallas import tpu as pltpu
from jax.experimental.pallas import tpu_sc as plsc

_NC, _NS, _L = 2, 16, 16
_NW = _NC * _NS


def _sc_body(x_hbm, pe_hbm, idx_hbm, out_hbm,
             idx_v, rows_v, xbuf, obuf, sem_pe, sem_in, sem_out):
    var_num, batch, d = x_hbm.shape
    bw = batch // _NW
    nidx = idx_hbm.shape[1]
    nvec = d // _L
    wid = lax.axis_index("s") * _NC + lax.axis_index("c")
    base = wid * bw

    for j in range(idx_hbm.shape[0]):
        pltpu.sync_copy(idx_hbm.at[j], idx_v.at[j])
        pltpu.async_copy(pe_hbm.at[idx_v.at[j]],
                         rows_v.at[pl.ds(j * nidx, nidx)], sem_pe).wait()

    pltpu.async_copy(x_hbm.at[0, pl.ds(base, bw)], xbuf.at[0], sem_in.at[0])

    def step(iv, carry):
        for b in range(2):
            v = iv * 2 + b

            @pl.when(v + 1 < var_num)
            def _():
                pltpu.async_copy(x_hbm.at[v + 1, pl.ds(base, bw)],
                                 xbuf.at[1 - b], sem_in.at[1 - b])

            pltpu.make_async_copy(x_hbm.at[v, pl.ds(base, bw)],
                                  xbuf.at[b], sem_in.at[b]).wait()

            @pl.when(v >= 2)
            def _():
                pltpu.make_async_copy(obuf.at[b],
                                      out_hbm.at[v - 2, pl.ds(base, bw)],
                                      sem_out.at[b]).wait()

            pe_vecs = [rows_v[v, pl.ds(i * _L, _L)] for i in range(nvec)]
            for r in range(bw):
                for i in range(nvec):
                    obuf[b, r, pl.ds(i * _L, _L)] = (
                        xbuf[b, r, pl.ds(i * _L, _L)] + pe_vecs[i])

            pltpu.async_copy(obuf.at[b], out_hbm.at[v, pl.ds(base, bw)],
                             sem_out.at[b])
        return carry

    lax.fori_loop(0, var_num // 2, step, 0)

    for b in range(2):
        pltpu.make_async_copy(obuf.at[b],
                              out_hbm.at[var_num - 2 + b, pl.ds(base, bw)],
                              sem_out.at[b]).wait()


def kernel(x, pe, source_encoding):
    var_num, batch, d_model = x.shape
    pe2d = pe[0]
    nidx = 112
    idx_pad = jnp.concatenate(
        [source_encoding,
         jnp.zeros((2 * nidx - var_num,), jnp.int32)]).reshape(2, nidx)
    bw = batch // _NW
    run = pl.kernel(
        _sc_body,
        mesh=plsc.VectorSubcoreMesh(core_axis_name="c", subcore_axis_name="s"),
        out_type=jax.ShapeDtypeStruct(x.shape, x.dtype),
        scratch_types=[
            pltpu.VMEM((2, nidx), jnp.int32),
            pltpu.VMEM((2 * nidx, d_model), jnp.float32),
            pltpu.VMEM((2, bw, d_model), jnp.float32),
            pltpu.VMEM((2, bw, d_model), jnp.float32),
            pltpu.SemaphoreType.DMA,
            pltpu.SemaphoreType.DMA((2,)),
            pltpu.SemaphoreType.DMA((2,)),
        ],
    )
    return run(x, pe2d, idx_pad)


# speedup vs baseline: 2.0090x; 2.0090x over previous
import jax
import jax.numpy as jnp
from jax import lax
from jax.experimental import pallas as pl
from jax.experimental.pallas import tpu as pltpu
from jax.experimental.pallas import tpu_sc as plsc

_NC, _NS, _L = 2, 16, 16
_NW = _NC * _NS
_ROWS_PER_STEP = 20


def _sc_gather_body(pe_hbm, idx_hbm, tab_hbm, idx_v, rows_v, sem):
    nchunk, nidx = idx_hbm.shape
    wid = lax.axis_index("s") * _NC + lax.axis_index("c")

    @pl.when(wid < nchunk)
    def _():
        pltpu.sync_copy(idx_hbm.at[wid], idx_v)
        pltpu.async_copy(pe_hbm.at[idx_v], rows_v, sem).wait()
        pltpu.sync_copy(rows_v, tab_hbm.at[pl.ds(wid * nidx, nidx)])


def _tc_add_body(x_ref, tab_ref, o_ref):
    o_ref[...] = x_ref[...] + tab_ref[...]


def kernel(x, pe, source_encoding):
    var_num, batch, d_model = x.shape
    pe2d = pe[0]
    nidx = 112
    idx_pad = jnp.concatenate(
        [source_encoding,
         jnp.zeros((2 * nidx - var_num,), jnp.int32)]).reshape(2, nidx)

    gather = pl.kernel(
        _sc_gather_body,
        mesh=plsc.VectorSubcoreMesh(core_axis_name="c", subcore_axis_name="s"),
        out_type=jax.ShapeDtypeStruct((2 * nidx, d_model), jnp.float32),
        scratch_types=[
            pltpu.VMEM((nidx,), jnp.int32),
            pltpu.VMEM((nidx, d_model), jnp.float32),
            pltpu.SemaphoreType.DMA,
        ],
    )
    tab = gather(pe2d, idx_pad).reshape(2 * nidx, 1, d_model)

    return pl.pallas_call(
        _tc_add_body,
        grid=(var_num // _ROWS_PER_STEP,),
        in_specs=[
            pl.BlockSpec((_ROWS_PER_STEP, batch, d_model),
                         lambda i: (i, 0, 0)),
            pl.BlockSpec((_ROWS_PER_STEP, 1, d_model), lambda i: (i, 0, 0)),
        ],
        out_specs=pl.BlockSpec((_ROWS_PER_STEP, batch, d_model),
                               lambda i: (i, 0, 0)),
        out_shape=jax.ShapeDtypeStruct(x.shape, x.dtype),
    )(x, tab)
